# Initial kernel scaffold; baseline (speedup 1.0000x reference)
import jax
import jax.numpy as jnp
from jax.experimental import pallas as pl


def kernel(loc, conf, defaultbox, target):
    raise NotImplementedError("write your pallas kernel here")


# speedup vs baseline: 6.1612x; 6.1612x over previous
import functools

import jax
import jax.numpy as jnp
from jax.experimental import pallas as pl

B, NBOX, NCLS, NT = 32, 20000, 81, 50
IOU_THRESH = 0.5
NT_PAD = 128
BN = 1000


def _ssd_block(db_ref, tT_ref, mb_ref, lc_ref):
    db = db_ref[...]
    cx = db[:, 0:1]
    cy = db[:, 1:2]
    hw = db[:, 2:3] * 0.5
    hh = db[:, 3:4] * 0.5
    dx0 = cx - hw
    dy0 = cy - hh
    dx1 = cx + hw
    dy1 = cy + hh
    area_d = (dx1 - dx0) * (dy1 - dy0)

    t = tT_ref[0]
    tx0 = t[0:1, :]
    ty0 = t[1:2, :]
    tx1 = t[2:3, :]
    ty1 = t[3:4, :]
    area_t = (tx1 - tx0) * (ty1 - ty0)

    iw = jnp.clip(jnp.minimum(dx1, tx1) - jnp.maximum(dx0, tx0), 0.0)
    ih = jnp.clip(jnp.minimum(dy1, ty1) - jnp.maximum(dy0, ty0), 0.0)
    inter = iw * ih
    union = jnp.maximum(area_d + area_t - inter, 1e-9)
    iou = inter / union

    col = jax.lax.broadcasted_iota(jnp.int32, iou.shape, 1)
    iou = jnp.where(col < NT, iou, -1.0)
    best_iou = jnp.max(iou, axis=1, keepdims=True)
    best = jnp.min(jnp.where(iou == best_iou, col, NT_PAD),
                   axis=1, keepdims=True)
    onehot = (col == best).astype(jnp.float32)

    sel = jax.lax.dot_general(
        onehot, t[0:5, :], (((1,), (1,)), ((), ())),
        precision=jax.lax.Precision.HIGHEST,
        preferred_element_type=jnp.float32)
    lc_ref[0] = sel[:, 0:4]

    cls = jnp.where(best_iou >= IOU_THRESH, sel[:, 4:5], 0.0).astype(jnp.int32)
    ccol = jax.lax.broadcasted_iota(jnp.int32, (db.shape[0], NCLS), 1)
    mb_ref[0] = (ccol == cls).astype(jnp.int32)


@jax.jit
def kernel(loc, conf, defaultbox, target):
    del loc, conf
    tT = jnp.transpose(target, (0, 2, 1))
    tT = jnp.pad(tT, ((0, 0), (0, 8 - tT.shape[1]), (0, NT_PAD - NT)))

    grid = (B, NBOX // BN)
    matchbox, loc_conf = pl.pallas_call(
        _ssd_block,
        grid=grid,
        in_specs=[
            pl.BlockSpec((BN, 4), lambda i, j: (j, 0)),
            pl.BlockSpec((1, 8, NT_PAD), lambda i, j: (i, 0, 0)),
        ],
        out_specs=[
            pl.BlockSpec((1, BN, NCLS), lambda i, j: (i, j, 0)),
            pl.BlockSpec((1, BN, 4), lambda i, j: (i, j, 0)),
        ],
        out_shape=[
            jax.ShapeDtypeStruct((B, NBOX, NCLS), jnp.int32),
            jax.ShapeDtypeStruct((B, NBOX, 4), jnp.float32),
        ],
    )(defaultbox, tT)
    return matchbox, loc_conf


# speedup vs baseline: 15.4169x; 2.5022x over previous
import jax
import jax.numpy as jnp
from jax.experimental import pallas as pl

B, NBOX, NCLS, NT = 32, 20000, 81, 50
IOU_THRESH = 0.5
NTP = 56
BN = 2048


def _ssd_block(db_ref, t_ref, mb_ref, lc_ref):
    t = t_ref[0]
    tx0 = t[:, 0:1]
    ty0 = t[:, 1:2]
    tx1 = t[:, 2:3]
    ty1 = t[:, 3:4]
    area_t = (tx1 - tx0) * (ty1 - ty0)

    db = db_ref[...]
    hw = db[2:3, :] * 0.5
    hh = db[3:4, :] * 0.5
    dx0 = db[0:1, :] - hw
    dx1 = db[0:1, :] + hw
    dy0 = db[1:2, :] - hh
    dy1 = db[1:2, :] + hh
    area_d = (dx1 - dx0) * (dy1 - dy0)

    iw = jnp.clip(jnp.minimum(dx1, tx1) - jnp.maximum(dx0, tx0), 0.0)
    ih = jnp.clip(jnp.minimum(dy1, ty1) - jnp.maximum(dy0, ty0), 0.0)
    inter = iw * ih
    iou = inter / jnp.maximum(area_d + area_t - inter, 1e-9)

    best_iou = jnp.max(iou, axis=0, keepdims=True)
    row = jax.lax.broadcasted_iota(jnp.int32, iou.shape, 0)
    best = jnp.min(jnp.where(iou == best_iou, row, NTP),
                   axis=0, keepdims=True)
    onehot = (row == best).astype(jnp.float32)
    keep = (best_iou >= IOU_THRESH).astype(jnp.float32)

    dn = (((0,), (0,)), ((), ()))
    m1 = jax.lax.dot_general(t, onehot, dn,
                             precision=jax.lax.Precision.HIGHEST,
                             preferred_element_type=jnp.float32)
    m2 = jax.lax.dot_general(t, onehot * keep, dn,
                             precision=jax.lax.Precision.HIGHEST,
                             preferred_element_type=jnp.float32)
    sel = jnp.concatenate([m1[0:4, :], m2[4:8, :]], axis=0)
    tr = jnp.transpose(sel)
    lc_ref[0] = tr[:, 0:4]

    cls = tr[:, 4:5].astype(jnp.int32)
    ccol = jax.lax.broadcasted_iota(jnp.int32, (tr.shape[0], NCLS), 1)
    mb_ref[0] = (ccol == cls).astype(jnp.int32)


@jax.jit
def kernel(loc, conf, defaultbox, target):
    del loc, conf
    dbT = jnp.transpose(defaultbox)
    tp = jnp.pad(target, ((0, 0), (0, NTP - NT), (0, 8 - target.shape[-1])))

    grid = (B, pl.cdiv(NBOX, BN))
    matchbox, loc_conf = pl.pallas_call(
        _ssd_block,
        grid=grid,
        in_specs=[
            pl.BlockSpec((4, BN), lambda i, j: (0, j)),
            pl.BlockSpec((1, NTP, 8), lambda i, j: (i, 0, 0)),
        ],
        out_specs=[
            pl.BlockSpec((1, BN, NCLS), lambda i, j: (i, j, 0)),
            pl.BlockSpec((1, BN, 4), lambda i, j: (i, j, 0)),
        ],
        out_shape=[
            jax.ShapeDtypeStruct((B, NBOX, NCLS), jnp.int32),
            jax.ShapeDtypeStruct((B, NBOX, 4), jnp.float32),
        ],
    )(dbT, tp)
    return matchbox, loc_conf


# speedup vs baseline: 16.9425x; 1.0990x over previous
import jax
import jax.numpy as jnp
from jax.experimental import pallas as pl

B, NBOX, NCLS, NT = 32, 20000, 81, 50
IOU_THRESH = 0.5
NTP = 56
BN = 2048


def _ssd_block(db_ref, t_ref, mb_ref, lc_ref):
    t = t_ref[0]
    tx0 = t[:, 0:1]
    ty0 = t[:, 1:2]
    tx1 = t[:, 2:3]
    ty1 = t[:, 3:4]
    area_t = (tx1 - tx0) * (ty1 - ty0)

    db = db_ref[...]
    hw = db[2:3, :] * 0.5
    hh = db[3:4, :] * 0.5
    dx0 = db[0:1, :] - hw
    dx1 = db[0:1, :] + hw
    dy0 = db[1:2, :] - hh
    dy1 = db[1:2, :] + hh
    area_d = (dx1 - dx0) * (dy1 - dy0)

    iw = jnp.clip(jnp.minimum(dx1, tx1) - jnp.maximum(dx0, tx0), 0.0)
    ih = jnp.clip(jnp.minimum(dy1, ty1) - jnp.maximum(dy0, ty0), 0.0)
    inter = iw * ih
    iou = inter / jnp.maximum(area_d + area_t - inter, 1e-9)

    best_iou = jnp.max(iou, axis=0, keepdims=True)
    row = jax.lax.broadcasted_iota(jnp.int32, iou.shape, 0)
    best = jnp.min(jnp.where(iou == best_iou, row, NTP),
                   axis=0, keepdims=True)
    onehot = (row == best).astype(jnp.float32)
    keep = (best_iou >= IOU_THRESH).astype(jnp.float32)

    dn = (((0,), (0,)), ((), ()))
    m1 = jax.lax.dot_general(t, onehot, dn,
                             precision=jax.lax.Precision.HIGHEST,
                             preferred_element_type=jnp.float32)
    cls_row = m1[4:5, :] * keep
    sel = jnp.concatenate([m1[0:4, :], cls_row, m1[5:8, :]], axis=0)
    tr = jnp.transpose(sel)
    lc_ref[0] = tr[:, 0:4]

    cls = tr[:, 4:5].astype(jnp.int32)
    ccol = jax.lax.broadcasted_iota(jnp.int32, (tr.shape[0], NCLS), 1)
    mb_ref[0] = (ccol == cls).astype(jnp.int32)


@jax.jit
def kernel(loc, conf, defaultbox, target):
    del loc, conf
    dbT = jnp.transpose(defaultbox)
    tp = jnp.pad(target, ((0, 0), (0, NTP - NT), (0, 8 - target.shape[-1])))

    grid = (B, pl.cdiv(NBOX, BN))
    matchbox, loc_conf = pl.pallas_call(
        _ssd_block,
        grid=grid,
        in_specs=[
            pl.BlockSpec((4, BN), lambda i, j: (0, j)),
            pl.BlockSpec((1, NTP, 8), lambda i, j: (i, 0, 0)),
        ],
        out_specs=[
            pl.BlockSpec((1, BN, NCLS), lambda i, j: (i, j, 0)),
            pl.BlockSpec((1, BN, 4), lambda i, j: (i, j, 0)),
        ],
        out_shape=[
            jax.ShapeDtypeStruct((B, NBOX, NCLS), jnp.int32),
            jax.ShapeDtypeStruct((B, NBOX, 4), jnp.float32),
        ],
    )(dbT, tp)
    return matchbox, loc_conf


# speedup vs baseline: 18.9293x; 1.1173x over previous
import jax
import jax.numpy as jnp
from jax.experimental import pallas as pl

B, NBOX, NCLS, NT = 32, 20000, 81, 50
IOU_THRESH = 0.5
NTP = 56
BN = 4096


def _ssd_block(db_ref, t_ref, mb_ref, lc_ref):
    t = t_ref[0]
    tx0 = t[:, 0:1]
    ty0 = t[:, 1:2]
    tx1 = t[:, 2:3]
    ty1 = t[:, 3:4]
    area_t = (tx1 - tx0) * (ty1 - ty0)

    db = db_ref[...]
    hw = db[2:3, :] * 0.5
    hh = db[3:4, :] * 0.5
    dx0 = db[0:1, :] - hw
    dx1 = db[0:1, :] + hw
    dy0 = db[1:2, :] - hh
    dy1 = db[1:2, :] + hh
    area_d = (dx1 - dx0) * (dy1 - dy0)

    iw = jnp.clip(jnp.minimum(dx1, tx1) - jnp.maximum(dx0, tx0), 0.0)
    ih = jnp.clip(jnp.minimum(dy1, ty1) - jnp.maximum(dy0, ty0), 0.0)
    inter = iw * ih
    iou = inter / jnp.maximum(area_d + area_t - inter, 1e-9)

    best_iou = jnp.max(iou, axis=0, keepdims=True)
    row = jax.lax.broadcasted_iota(jnp.int32, iou.shape, 0)
    best = jnp.min(jnp.where(iou == best_iou, row, NTP),
                   axis=0, keepdims=True)
    onehot = (row == best).astype(jnp.float32)
    keep = (best_iou >= IOU_THRESH).astype(jnp.float32)

    dn = (((0,), (0,)), ((), ()))
    m1 = jax.lax.dot_general(t, onehot, dn,
                             precision=jax.lax.Precision.HIGHEST,
                             preferred_element_type=jnp.float32)
    cls_row = m1[4:5, :] * keep
    sel = jnp.concatenate([m1[0:4, :], cls_row, m1[5:8, :]], axis=0)
    tr = jnp.transpose(sel)
    lc_ref[0] = tr[:, 0:4]

    cls = tr[:, 4:5].astype(jnp.int32)
    ccol = jax.lax.broadcasted_iota(jnp.int32, (tr.shape[0], NCLS), 1)
    mb_ref[0] = (ccol == cls).astype(jnp.int32)


@jax.jit
def kernel(loc, conf, defaultbox, target):
    del loc, conf
    dbT = jnp.transpose(defaultbox)
    tp = jnp.pad(target, ((0, 0), (0, NTP - NT), (0, 8 - target.shape[-1])))

    grid = (B, pl.cdiv(NBOX, BN))
    matchbox, loc_conf = pl.pallas_call(
        _ssd_block,
        grid=grid,
        in_specs=[
            pl.BlockSpec((4, BN), lambda i, j: (0, j)),
            pl.BlockSpec((1, NTP, 8), lambda i, j: (i, 0, 0)),
        ],
        out_specs=[
            pl.BlockSpec((1, BN, NCLS), lambda i, j: (i, j, 0)),
            pl.BlockSpec((1, BN, 4), lambda i, j: (i, j, 0)),
        ],
        out_shape=[
            jax.ShapeDtypeStruct((B, NBOX, NCLS), jnp.int32),
            jax.ShapeDtypeStruct((B, NBOX, 4), jnp.float32),
        ],
    )(dbT, tp)
    return matchbox, loc_conf
